# baseline (speedup 1.0000x reference)
import math

import jax
import jax.numpy as jnp
from jax.experimental import pallas as pl
from jax.experimental.pallas import tpu as pltpu

_B, _T, _D = 1, 2048, 768
_E, _K = 8, 2
_H = 2048
_CAP = 1024
_AUX_W, _Z_W, _DROP_W = 0.01, 0.001, 0.001
_N = _B * _T
_NK = _N * _K

_TM = 256
_TN = 256


def _ffn_body(meta_ref, xf_ref, tokpad_ref, w13a_ref, w13b_ref,
              w2a_ref, w2b_ref, o_ref):
    e = pl.program_id(0)
    t = pl.program_id(1)
    cnt = jnp.minimum(meta_ref[e], _CAP)

    @pl.when(cnt > t * _TM)
    def _compute():
        idx = tokpad_ref[0, 0, pl.ds(t * _TM, _TM)]
        iota = jax.lax.broadcasted_iota(jnp.int32, (_TM, _N), 1)
        oh = jnp.where(iota == idx[:, None], 1.0, 0.0)
        xb = jnp.dot(oh, xf_ref[...], preferred_element_type=jnp.float32)
        z = xb + xb
        gu = (jnp.dot(z[:, :_D // 2], w13a_ref[0],
                      preferred_element_type=jnp.float32) +
              jnp.dot(z[:, _D // 2:], w13b_ref[0],
                      preferred_element_type=jnp.float32))
        act = jax.nn.silu(gu[:, :_H]) * gu[:, _H:]
        y = (xb +
             jnp.dot(act[:, :_H // 2], w2a_ref[0],
                     preferred_element_type=jnp.float32) +
             jnp.dot(act[:, _H // 2:], w2b_ref[0],
                     preferred_element_type=jnp.float32))
        o_ref[0] = y.astype(jnp.bfloat16)

    @pl.when(cnt <= t * _TM)
    def _skip():
        o_ref[0] = jnp.zeros((_TM, _D), jnp.bfloat16)


def _expert_ffn_pallas(meta, xf, tok_pad, w13, w2):
    grid_spec = pltpu.PrefetchScalarGridSpec(
        num_scalar_prefetch=1,
        grid=(_E, _CAP // _TM),
        in_specs=[
            pl.BlockSpec((_N, _D), lambda e, t, m: (0, 0)),
            pl.BlockSpec((1, 1, _CAP), lambda e, t, m: (e, 0, 0)),
            pl.BlockSpec((1, _D // 2, 2 * _H), lambda e, t, m: (e, 0, 0)),
            pl.BlockSpec((1, _D // 2, 2 * _H), lambda e, t, m: (e, 1, 0)),
            pl.BlockSpec((1, _H // 2, _D), lambda e, t, m: (e, 0, 0)),
            pl.BlockSpec((1, _H // 2, _D), lambda e, t, m: (e, 1, 0)),
        ],
        out_specs=pl.BlockSpec((1, _TM, _D), lambda e, t, m: (e, t, 0)),
    )
    return pl.pallas_call(
        _ffn_body,
        grid_spec=grid_spec,
        out_shape=jax.ShapeDtypeStruct((_E, _CAP, _D), jnp.bfloat16),
    )(meta, xf, tok_pad, w13, w13, w2, w2)


def _dot_t(a, w):
    return jax.lax.dot_general(a, w, (((1,), (1,)), ((), ())),
                               preferred_element_type=jnp.float32)


def _collab_body(sel_ref, aux_ref, msg_w_ref, q_w_ref, k_w_ref,
                 w1_ref, w2_ref, o_w_ref, out_ref):
    f32 = jnp.float32
    sel = (sel_ref[0].astype(f32), sel_ref[1].astype(f32))
    Ms = [_dot_t(s, msg_w_ref[...]) for s in sel]
    Qs = [_dot_t(s, q_w_ref[...]) for s in sel]
    Ks = [_dot_t(m, k_w_ref[...]) for m in Ms]
    kms = (aux_ref[:, 0:1], aux_ref[:, 1:2])
    gts = (aux_ref[:, 2:3], aux_ref[:, 3:4])
    inv = 1.0 / math.sqrt(_D)
    neg = jnp.finfo(f32).min

    def sc(i, j):
        raw = jnp.sum(Qs[i] * Ks[j], axis=-1, keepdims=True) * inv
        return jnp.where(kms[i] * kms[j] > 0, raw, neg)

    s = [[sc(i, j) for j in range(_K)] for i in range(_K)]
    ys = []
    for i in range(_K):
        m = jnp.maximum(s[i][0], s[i][1])
        e0 = jnp.exp(s[i][0] - m)
        e1 = jnp.exp(s[i][1] - m)
        dn = e0 + e1
        a0 = e0 / dn * kms[i]
        a1 = e1 / dn * kms[i]
        msg = a0 * Ms[0] + a1 * Ms[1]
        upd_in = jnp.concatenate([sel[i], msg], axis=-1)
        pre = _dot_t(upd_in, w1_ref[...])
        h1 = 0.5 * pre * (1.0 + jax.lax.erf(pre * (1.0 / math.sqrt(2.0))))
        h = _dot_t(h1, w2_ref[...])
        ys.append(gts[i] * (sel[i] + h))
    y_tok = ys[0] + ys[1]
    out_ref[...] = _dot_t(y_tok, o_w_ref[...])


def _collab_pallas(sel_de, aux_tok, msg_w, q_w, k_w, w1, w2, o_w):
    wspec = lambda shape: pl.BlockSpec(shape, lambda t: (0, 0))
    return pl.pallas_call(
        _collab_body,
        grid=(_N // _TN,),
        in_specs=[
            pl.BlockSpec((_K, _TN, _D), lambda t: (0, t, 0)),
            pl.BlockSpec((_TN, 8), lambda t: (t, 0)),
            wspec((_D, _D)),
            wspec((_D, _D)),
            wspec((_D, _D)),
            wspec((2 * _D, 2 * _D)),
            wspec((_D, 2 * _D)),
            wspec((_D, _D)),
        ],
        out_specs=pl.BlockSpec((_TN, _D), lambda t: (t, 0)),
        out_shape=jax.ShapeDtypeStruct((_N, _D), jnp.float32),
    )(sel_de, aux_tok, msg_w, q_w, k_w, w1, w2, o_w)


def kernel(x, gate_w, w13, w2, msg_w, q_w, k_w, upd_w1, upd_w2, o_w):
    xf = x.reshape(_N, _D)
    logits = xf @ gate_w.T
    i1 = jnp.argmax(logits, axis=-1)
    v1 = jnp.max(logits, axis=-1)
    eidx = jnp.arange(_E)[None, :]
    masked = jnp.where(eidx == i1[:, None], -jnp.inf, logits)
    i2 = jnp.argmax(masked, axis=-1)
    v2 = jnp.max(masked, axis=-1)
    topk_vals = jnp.stack([v1, v2], axis=-1)
    topk_idx = jnp.stack([i1, i2], axis=-1).astype(jnp.int32)
    topk_probs = jax.nn.softmax(topk_vals, axis=-1)
    router_probs = jax.nn.softmax(logits, axis=-1)
    onehot = (topk_idx[:, :, None] == jnp.arange(_E)[None, None, :])
    assign_mean = (topk_probs[:, :, None] * onehot).sum(axis=(0, 1)) / _N
    balance = (router_probs.mean(0) * assign_mean).sum() * _E
    zlse = jax.nn.logsumexp(logits, axis=-1)
    aux = _AUX_W * balance + _Z_W * (zlse * zlse).mean()

    target = topk_idx.reshape(-1)
    prio = topk_vals.reshape(-1)
    tok_ids = jnp.repeat(jnp.arange(_N, dtype=jnp.int32), _K)
    row_ids = jnp.arange(_NK, dtype=jnp.int32)
    _, _, tok_sorted, order = jax.lax.sort(
        (target, -prio, tok_ids, row_ids), num_keys=2, is_stable=True)
    onehot_t = (target[:, None] == jnp.arange(_E)[None, :])
    counts = onehot_t.sum(0, dtype=jnp.int32)
    starts = jnp.concatenate(
        [jnp.zeros((1,), jnp.int32), jnp.cumsum(counts)[:-1]])

    tok_sp = jnp.concatenate([tok_sorted, jnp.full((_CAP,), -1, jnp.int32)])
    c_grid = jnp.arange(_CAP)[None, :]
    caps = jnp.minimum(counts, _CAP)
    segs = [jax.lax.dynamic_slice(tok_sp, (starts[e],), (_CAP,))
            for e in range(_E)]
    tok_cap = jnp.where(c_grid < caps[:, None], jnp.stack(segs), -1)
    meta = jnp.concatenate([counts, starts])

    y_cap = _expert_ffn_pallas(meta, xf, tok_cap.reshape(_E, 1, _CAP),
                               w13, w2)

    inv_order = jnp.argsort(order).astype(jnp.int32)
    starts_row = jnp.sum(jnp.where(onehot_t, starts[None, :], 0), axis=-1)
    wr = inv_order - starts_row
    keptr = wr < _CAP
    slot = target * _CAP + jnp.clip(wr, 0, _CAP - 1)
    y_flat = y_cap.reshape(_E * _CAP, _D)
    slot_de = slot.reshape(_N, _K).T.reshape(-1)
    sel_de = y_flat[slot_de].reshape(_K, _N, _D)

    drop_frac = 1.0 - keptr.astype(jnp.float32).mean()
    aux = aux + _DROP_W * drop_frac

    km = keptr.reshape(_N, _K).astype(jnp.float32)
    gts = topk_probs * km
    aux_tok = jnp.concatenate(
        [km, gts, jnp.zeros((_N, 4), jnp.float32)], axis=1)

    y = _collab_pallas(sel_de, aux_tok, msg_w, q_w, k_w,
                       upd_w1, upd_w2, o_w)
    return y.reshape(_B, _T, _D), aux, topk_idx.reshape(_B, _T, _K)


# speedup vs baseline: 1.0620x; 1.0620x over previous
import math

import jax
import jax.numpy as jnp
from jax.experimental import pallas as pl
from jax.experimental.pallas import tpu as pltpu

_B, _T, _D = 1, 2048, 768
_E, _K = 8, 2
_H = 2048
_CAP = 1024
_AUX_W, _Z_W, _DROP_W = 0.01, 0.001, 0.001
_N = _B * _T
_NK = _N * _K

_TM = 256
_TN = 512


def _ffn_body(meta_ref, xf_ref, tokpad_ref, w13_ref, w2_ref, o_ref):
    e = pl.program_id(0)
    t = pl.program_id(1)
    cnt = jnp.minimum(meta_ref[e], _CAP)

    @pl.when(cnt > t * _TM)
    def _compute():
        idx = tokpad_ref[0, 0, pl.ds(t * _TM, _TM)]
        iota = jax.lax.broadcasted_iota(jnp.int32, (_TM, _N), 1)
        oh = jnp.where(iota == idx[:, None], 1.0, 0.0)
        xb = jnp.dot(oh, xf_ref[...], preferred_element_type=jnp.float32)
        z = xb + xb
        gu = jnp.dot(z, w13_ref[0], preferred_element_type=jnp.float32)
        act = jax.nn.silu(gu[:, :_H]) * gu[:, _H:]
        y = xb + jnp.dot(act, w2_ref[0], preferred_element_type=jnp.float32)
        o_ref[0] = y.astype(jnp.bfloat16)

    @pl.when(cnt <= t * _TM)
    def _skip():
        o_ref[0] = jnp.zeros((_TM, _D), jnp.bfloat16)


def _expert_ffn_pallas(meta, xf, tok_pad, w13, w2):
    grid_spec = pltpu.PrefetchScalarGridSpec(
        num_scalar_prefetch=1,
        grid=(_E, _CAP // _TM),
        in_specs=[
            pl.BlockSpec((_N, _D), lambda e, t, m: (0, 0)),
            pl.BlockSpec((1, 1, _CAP), lambda e, t, m: (e, 0, 0)),
            pl.BlockSpec((1, _D, 2 * _H), lambda e, t, m: (e, 0, 0)),
            pl.BlockSpec((1, _H, _D), lambda e, t, m: (e, 0, 0)),
        ],
        out_specs=pl.BlockSpec((1, _TM, _D), lambda e, t, m: (e, t, 0)),
    )
    return pl.pallas_call(
        _ffn_body,
        grid_spec=grid_spec,
        out_shape=jax.ShapeDtypeStruct((_E, _CAP, _D), jnp.bfloat16),
    )(meta, xf, tok_pad, w13, w2)


def _dot_t(a, w):
    return jax.lax.dot_general(a, w, (((1,), (1,)), ((), ())),
                               preferred_element_type=jnp.float32)


def _collab_body(sel_ref, aux_ref, msg_w_ref, q_w_ref, k_w_ref,
                 w1_ref, w2_ref, o_w_ref, out_ref):
    f32 = jnp.float32
    sel = (sel_ref[0].astype(f32), sel_ref[1].astype(f32))
    Ms = [_dot_t(s, msg_w_ref[...]) for s in sel]
    Qs = [_dot_t(s, q_w_ref[...]) for s in sel]
    Ks = [_dot_t(m, k_w_ref[...]) for m in Ms]
    kms = (aux_ref[:, 0:1], aux_ref[:, 1:2])
    gts = (aux_ref[:, 2:3], aux_ref[:, 3:4])
    inv = 1.0 / math.sqrt(_D)
    neg = jnp.finfo(f32).min

    def sc(i, j):
        raw = jnp.sum(Qs[i] * Ks[j], axis=-1, keepdims=True) * inv
        return jnp.where(kms[i] * kms[j] > 0, raw, neg)

    s = [[sc(i, j) for j in range(_K)] for i in range(_K)]
    ys = []
    for i in range(_K):
        m = jnp.maximum(s[i][0], s[i][1])
        e0 = jnp.exp(s[i][0] - m)
        e1 = jnp.exp(s[i][1] - m)
        dn = e0 + e1
        a0 = e0 / dn * kms[i]
        a1 = e1 / dn * kms[i]
        msg = a0 * Ms[0] + a1 * Ms[1]
        upd_in = jnp.concatenate([sel[i], msg], axis=-1)
        pre = _dot_t(upd_in, w1_ref[...])
        h1 = 0.5 * pre * (1.0 + jax.lax.erf(pre * (1.0 / math.sqrt(2.0))))
        h = _dot_t(h1, w2_ref[...])
        ys.append(gts[i] * (sel[i] + h))
    y_tok = ys[0] + ys[1]
    out_ref[...] = _dot_t(y_tok, o_w_ref[...])


def _collab_pallas(sel_de, aux_tok, msg_w, q_w, k_w, w1, w2, o_w):
    wspec = lambda shape: pl.BlockSpec(shape, lambda t: (0, 0))
    return pl.pallas_call(
        _collab_body,
        grid=(_N // _TN,),
        in_specs=[
            pl.BlockSpec((_K, _TN, _D), lambda t: (0, t, 0)),
            pl.BlockSpec((_TN, 8), lambda t: (t, 0)),
            wspec((_D, _D)),
            wspec((_D, _D)),
            wspec((_D, _D)),
            wspec((2 * _D, 2 * _D)),
            wspec((_D, 2 * _D)),
            wspec((_D, _D)),
        ],
        out_specs=pl.BlockSpec((_TN, _D), lambda t: (t, 0)),
        out_shape=jax.ShapeDtypeStruct((_N, _D), jnp.float32),
    )(sel_de, aux_tok, msg_w, q_w, k_w, w1, w2, o_w)


def kernel(x, gate_w, w13, w2, msg_w, q_w, k_w, upd_w1, upd_w2, o_w):
    xf = x.reshape(_N, _D)
    logits = xf @ gate_w.T
    i1 = jnp.argmax(logits, axis=-1)
    v1 = jnp.max(logits, axis=-1)
    eidx = jnp.arange(_E)[None, :]
    masked = jnp.where(eidx == i1[:, None], -jnp.inf, logits)
    i2 = jnp.argmax(masked, axis=-1)
    v2 = jnp.max(masked, axis=-1)
    topk_vals = jnp.stack([v1, v2], axis=-1)
    topk_idx = jnp.stack([i1, i2], axis=-1).astype(jnp.int32)
    topk_probs = jax.nn.softmax(topk_vals, axis=-1)
    router_probs = jax.nn.softmax(logits, axis=-1)
    onehot = (topk_idx[:, :, None] == jnp.arange(_E)[None, None, :])
    assign_mean = (topk_probs[:, :, None] * onehot).sum(axis=(0, 1)) / _N
    balance = (router_probs.mean(0) * assign_mean).sum() * _E
    zlse = jax.nn.logsumexp(logits, axis=-1)
    aux = _AUX_W * balance + _Z_W * (zlse * zlse).mean()

    target = topk_idx.reshape(-1)
    prio = topk_vals.reshape(-1)
    tok_ids = jnp.repeat(jnp.arange(_N, dtype=jnp.int32), _K)
    row_ids = jnp.arange(_NK, dtype=jnp.int32)
    _, _, tok_sorted, order = jax.lax.sort(
        (target, -prio, tok_ids, row_ids), num_keys=2, is_stable=True)
    onehot_t = (target[:, None] == jnp.arange(_E)[None, :])
    counts = onehot_t.sum(0, dtype=jnp.int32)
    starts = jnp.concatenate(
        [jnp.zeros((1,), jnp.int32), jnp.cumsum(counts)[:-1]])

    tok_sp = jnp.concatenate([tok_sorted, jnp.full((_CAP,), -1, jnp.int32)])
    c_grid = jnp.arange(_CAP)[None, :]
    caps = jnp.minimum(counts, _CAP)
    segs = [jax.lax.dynamic_slice(tok_sp, (starts[e],), (_CAP,))
            for e in range(_E)]
    tok_cap = jnp.where(c_grid < caps[:, None], jnp.stack(segs), -1)
    meta = jnp.concatenate([counts, starts])

    y_cap = _expert_ffn_pallas(meta, xf, tok_cap.reshape(_E, 1, _CAP),
                               w13, w2)

    inv_order = jnp.argsort(order).astype(jnp.int32)
    starts_row = jnp.sum(jnp.where(onehot_t, starts[None, :], 0), axis=-1)
    wr = inv_order - starts_row
    keptr = wr < _CAP
    slot = target * _CAP + jnp.clip(wr, 0, _CAP - 1)
    y_flat = y_cap.reshape(_E * _CAP, _D)
    slot_de = slot.reshape(_N, _K).T.reshape(-1)
    sel_de = y_flat[slot_de].reshape(_K, _N, _D)

    drop_frac = 1.0 - keptr.astype(jnp.float32).mean()
    aux = aux + _DROP_W * drop_frac

    km = keptr.reshape(_N, _K).astype(jnp.float32)
    gts = topk_probs * km
    aux_tok = jnp.concatenate(
        [km, gts, jnp.zeros((_N, 4), jnp.float32)], axis=1)

    y = _collab_pallas(sel_de, aux_tok, msg_w, q_w, k_w,
                       upd_w1, upd_w2, o_w)
    return y.reshape(_B, _T, _D), aux, topk_idx.reshape(_B, _T, _K)
